# baseline (speedup 1.0000x reference)
import functools
import numpy as np
import jax
import jax.numpy as jnp
from jax import lax
from jax.experimental import pallas as pl
from jax.experimental.pallas import tpu as pltpu
from jax.experimental.pallas import tpu_sc as plsc

_N = 10000
_E = 320000
_C = 128
_DEG = 32
_DELTA = float(np.log(_DEG + 1.0))

_NC = 2
_NS = 16
_L = 16
_NW = _NC * _NS

_NB = 64
_BRNG = 157
_BRP = 160
_GARB = _BRNG
_NPAD = _NB * _BRNG
_WRNG = 2 * _BRNG
_WRP = 320
_CH = 8000
_STG = _CH + 16
_K = 64
_NBUF = 4
_IB = 4096
_ECAP = _E + 16384
_BIG = 3.0e38

_ROWS = 400

_mesh = plsc.VectorSubcoreMesh(core_axis_name="c", subcore_axis_name="s")
_params = pltpu.CompilerParams(needs_layout_passes=False,
                               use_tc_tiling_on_sc=False)


def _bin_body(src_hbm, dst_hbm, lk_hbm, cnt_hbm, deg_hbm,
              srcvs, dstvs, st0, st1, degv, cntv, sems):
    cid = lax.axis_index("c")
    sid = lax.axis_index("s")
    w = cid * _NS + sid
    q0 = 2 * w
    q1 = 2 * w + 1

    zeros16f = jnp.zeros((_L,), jnp.float32)
    for j in range(_WRP // _L):
        degv[pl.ds(j * _L, _L)] = zeros16f

    ones16 = jnp.ones((_L,), jnp.float32)
    lanes = lax.iota(jnp.int32, _L)
    garb16 = jnp.full((_L,), _GARB, jnp.int32)

    def load_chunk(g, r):
        pltpu.async_copy(src_hbm.at[pl.ds(g * _CH, _CH)], srcvs[r], sems[2 * r])
        pltpu.async_copy(dst_hbm.at[pl.ds(g * _CH, _CH)], dstvs[r],
                         sems[2 * r + 1])

    def wait_chunk(g, r):
        pltpu.make_async_copy(src_hbm.at[pl.ds(g * _CH, _CH)], srcvs[r],
                              sems[2 * r]).wait()
        pltpu.make_async_copy(dst_hbm.at[pl.ds(g * _CH, _CH)], dstvs[r],
                              sems[2 * r + 1]).wait()

    def scan_chunk(r, tots):
        tot0, tot1 = tots
        srcv = srcvs[r]
        dstv = dstvs[r]

        def vreg_body(j, cnts):
            c0, c1 = cnts
            s = srcv[pl.ds(j * _L, _L)]
            d = dstv[pl.ds(j * _L, _L)]
            b = d // _BRNG
            dl = d - b * _BRNG
            key = s * 256 + dl
            m0 = b == q0
            m1 = b == q1
            plsc.store_compressed(st0.at[pl.ds(c0, _L)], key, mask=m0)
            plsc.store_compressed(st1.at[pl.ds(c1, _L)], key, mask=m1)
            didx = d - w * _WRNG
            plsc.addupdate_scatter(degv, [didx], ones16, mask=m0 | m1)
            return (c0 + plsc.all_reduce_population_count(m0)[0],
                    c1 + plsc.all_reduce_population_count(m1)[0])

        c0, c1 = plsc.parallel_loop(
            0, _CH // _L, carry=(jnp.int32(0), jnp.int32(0)))(vreg_body)
        p0 = (8 - (c0 % 8)) % 8
        plsc.store_compressed(st0.at[pl.ds(c0, _L)], garb16, mask=lanes < p0)
        c0 = c0 + p0
        p1 = (8 - (c1 % 8)) % 8
        plsc.store_compressed(st1.at[pl.ds(c1, _L)], garb16, mask=lanes < p1)
        c1 = c1 + p1
        f0 = pl.multiple_of(q0 * _ECAP + tot0, 8)
        pltpu.sync_copy(st0, lk_hbm.at[pl.ds(f0, _STG)])
        f1 = pl.multiple_of(q1 * _ECAP + tot1, 8)
        pltpu.sync_copy(st1, lk_hbm.at[pl.ds(f1, _STG)])
        return (tot0 + c0, tot1 + c1)

    def pair_body(t, tots):
        g = 2 * t
        load_chunk(g + 1, 1)
        wait_chunk(g, 0)
        tots = scan_chunk(0, tots)

        @pl.when(g + 2 < _E // _CH)
        def _next():
            load_chunk(g + 2, 0)

        wait_chunk(g + 1, 1)
        return scan_chunk(1, tots)

    load_chunk(0, 0)
    tot0, tot1 = lax.fori_loop(0, _E // (2 * _CH), pair_body,
                               (jnp.int32(0), jnp.int32(0)))

    for j in range(_NBUF * _K // _L):
        st0[pl.ds(j * _L, _L)] = garb16
    g0 = pl.multiple_of(q0 * _ECAP + tot0, 8)
    pltpu.sync_copy(st0.at[pl.ds(0, _NBUF * _K)],
                    lk_hbm.at[pl.ds(g0, _NBUF * _K)])
    g1 = pl.multiple_of(q1 * _ECAP + tot1, 8)
    pltpu.sync_copy(st0.at[pl.ds(0, _NBUF * _K)],
                    lk_hbm.at[pl.ds(g1, _NBUF * _K)])

    nb0 = (tot0 + _K - 1) // _K
    nb0 = ((nb0 + _NBUF - 1) // _NBUF) * _NBUF
    nb1 = (tot1 + _K - 1) // _K
    nb1 = ((nb1 + _NBUF - 1) // _NBUF) * _NBUF
    cntv[...] = jnp.where(lanes == 0, nb0, 0)
    pltpu.sync_copy(cntv, cnt_hbm.at[q0])
    cntv[...] = jnp.where(lanes == 0, nb1, 0)
    pltpu.sync_copy(cntv, cnt_hbm.at[q1])
    pltpu.sync_copy(degv, deg_hbm.at[w])


_bin_edges = functools.partial(
    pl.kernel,
    out_type=[
        jax.ShapeDtypeStruct((_NB * _ECAP,), jnp.int32),
        jax.ShapeDtypeStruct((_NB, _L), jnp.int32),
        jax.ShapeDtypeStruct((_NW, _WRP), jnp.float32),
    ],
    mesh=_mesh,
    compiler_params=_params,
    scratch_types=[
        [pltpu.VMEM((_CH,), jnp.int32) for _ in range(2)],
        [pltpu.VMEM((_CH,), jnp.int32) for _ in range(2)],
        pltpu.VMEM((_STG,), jnp.int32),
        pltpu.VMEM((_STG,), jnp.int32),
        pltpu.VMEM((_WRP,), jnp.float32),
        pltpu.VMEM((_L,), jnp.int32),
        [pltpu.SemaphoreType.DMA for _ in range(4)],
    ],
)(_bin_body)


def _agg_body(x_hbm, lk_hbm, cnt_hbm,
              sum_hbm, ssq_hbm, mx_hbm, mn_hbm,
              kblk, gbufs, sidxs, sacc, qacc, mxa, mna, cntv, sems):
    cid = lax.axis_index("c")
    sid = lax.axis_index("s")
    w = cid * _NS + sid

    posbig = jnp.full((_L,), _BIG, jnp.float32)
    negbig = jnp.full((_L,), -_BIG, jnp.float32)
    zeros16 = jnp.zeros((_L,), jnp.float32)
    m255 = jnp.full((_L,), 255, jnp.int32)

    def issue(kloc, r):
        for j in range(_K // _L):
            sidxs[r][pl.ds(j * _L, _L)] = lax.shift_right_logical(
                kblk[pl.ds(kloc * _K + j * _L, _L)], 8)
        pltpu.async_copy(x_hbm.at[sidxs[r]], gbufs[r], sems[r])

    def drain_rmw(kloc, r):
        pltpu.make_async_copy(x_hbm.at[sidxs[r]], gbufs[r], sems[r]).wait()
        gbuf = gbufs[r]

        def edge16(jj, _):
            dvec = kblk[pl.ds(kloc * _K + jj * _L, _L)] & m255
            roffs = [dvec[l] * _C for l in range(_L)]
            ibase = jj * _L

            def chunk_upd(c):
                co = c * _L
                for l in range(_L):
                    v = gbuf[ibase + l, pl.ds(co, _L)]
                    o = pl.ds(roffs[l] + co, _L)
                    sacc[o] = sacc[o] + v
                    qacc[o] = qacc[o] + v * v
                    mxa[o] = jnp.maximum(mxa[o], v)
                    mna[o] = jnp.minimum(mna[o], v)

            plsc.parallel_loop(0, _C // _L)(chunk_upd)
            return 0
        lax.fori_loop(0, _K // _L, edge16, 0)

    def bin_loop(sub, _):
        q = 2 * w + sub
        base = q * _ECAP

        def init_body(j, _2):
            o = pl.ds(j * _L, _L)
            sacc[o] = zeros16
            qacc[o] = zeros16
            mxa[o] = negbig
            mna[o] = posbig
            return 0
        lax.fori_loop(0, _BRP * _C // _L, init_body, 0)

        pltpu.sync_copy(cnt_hbm.at[q], cntv)
        nbp = cntv[...][0]

        def block_loop(tI, _2):
            boff = pl.multiple_of(base + tI * _IB, 8)
            pltpu.sync_copy(lk_hbm.at[pl.ds(boff, _IB)], kblk)
            nrem = jnp.minimum(nbp - tI * (_IB // _K), _IB // _K)

            @pl.when(nrem > 0)
            def _prologue():
                for r in range(_NBUF):
                    issue(r, r)

            def quad_loop(t, _3):
                k = _NBUF * t
                for r in range(_NBUF):
                    drain_rmw(k + r, r)

                    @pl.when(k + r + _NBUF < nrem)
                    def _next():
                        issue(k + r + _NBUF, r)
                return 0

            lax.fori_loop(0, nrem // _NBUF, quad_loop, 0)
            return 0

        lax.fori_loop(0, (nbp + _IB // _K - 1) // (_IB // _K), block_loop, 0)

        pltpu.sync_copy(sacc, sum_hbm.at[q])
        pltpu.sync_copy(qacc, ssq_hbm.at[q])
        pltpu.sync_copy(mxa, mx_hbm.at[q])
        pltpu.sync_copy(mna, mn_hbm.at[q])
        return 0

    lax.fori_loop(0, 2, bin_loop, 0)


_aggregate_sc = functools.partial(
    pl.kernel,
    out_type=[jax.ShapeDtypeStruct((_NB, _BRP * _C), jnp.float32)
              for _ in range(4)],
    mesh=_mesh,
    compiler_params=_params,
    scratch_types=[
        pltpu.VMEM((_IB,), jnp.int32),
        [pltpu.VMEM((_K, _C), jnp.float32) for _ in range(_NBUF)],
        [pltpu.VMEM((_K,), jnp.int32) for _ in range(_NBUF)],
        pltpu.VMEM((_BRP * _C,), jnp.float32),
        pltpu.VMEM((_BRP * _C,), jnp.float32),
        pltpu.VMEM((_BRP * _C,), jnp.float32),
        pltpu.VMEM((_BRP * _C,), jnp.float32),
        pltpu.VMEM((_L,), jnp.int32),
        [pltpu.SemaphoreType.DMA for _ in range(_NBUF)],
    ],
)(_agg_body)


def _dense_body(do_relu, x_ref, s_ref, q_ref, mx_ref, mn_ref, deg_ref,
                w_ref, b_ref, o_ref):
    deg = deg_ref[...]
    degc = jnp.maximum(deg, 1.0)
    inv = 1.0 / degc
    s = s_ref[...]
    mean = s * inv
    var = jnp.maximum(q_ref[...] * inv - mean * mean, 0.0)
    std = jnp.sqrt(var + 1e-5)
    has = deg > 0.0
    mx = jnp.where(has, mx_ref[...], 0.0)
    mn = jnp.where(has, mn_ref[...], 0.0)
    logd = jnp.log(deg + 1.0)
    amp = logd * (1.0 / _DELTA)
    att = _DELTA / jnp.clip(logd, 1e-5, None)

    agg = jnp.concatenate([mean, mn, mx, std], axis=1)
    w = w_ref[...]
    out = jnp.dot(x_ref[...], w[0:_C], preferred_element_type=jnp.float32)
    out += jnp.dot(agg, w[_C:5 * _C], preferred_element_type=jnp.float32)
    out += amp * jnp.dot(agg, w[5 * _C:9 * _C], preferred_element_type=jnp.float32)
    out += att * jnp.dot(agg, w[9 * _C:13 * _C], preferred_element_type=jnp.float32)
    out += b_ref[...]
    if do_relu:
        out = jnp.maximum(out, 0.0)
    o_ref[...] = out


def _dense_stage(x, s, q, mx, mn, degf, W, b, do_relu):
    grid = _N // _ROWS
    row_spec = pl.BlockSpec((_ROWS, _C), lambda i: (i, 0))
    out = pl.pallas_call(
        functools.partial(_dense_body, do_relu),
        grid=(grid,),
        in_specs=[
            row_spec, row_spec, row_spec, row_spec, row_spec,
            pl.BlockSpec((_ROWS, 1), lambda i: (i, 0)),
            pl.BlockSpec((13 * _C, _C), lambda i: (0, 0)),
            pl.BlockSpec((1, _C), lambda i: (0, 0)),
        ],
        out_specs=row_spec,
        out_shape=jax.ShapeDtypeStruct((_N, _C), jnp.float32),
    )(x, s, q, mx, mn, degf, W, b)
    return out


def kernel(x, edge_index, W0, b0, W1, b1, W2, b2):
    src = edge_index[0]
    dst = edge_index[1]

    lk, cnts, deg_rows = _bin_edges(src, dst)
    deg = deg_rows[:, :_WRNG].reshape(_NW * _WRNG)[:_N]
    degf = deg.reshape(_N, 1)

    def unpad(a):
        return a.reshape(_NB, _BRP, _C)[:, :_BRNG].reshape(_NPAD, _C)[:_N]

    h = x
    for W, b, relu in ((W0, b0, True), (W1, b1, True), (W2, b2, False)):
        s, q, mxf, mnf = _aggregate_sc(h, lk, cnts)
        h = _dense_stage(h, unpad(s), unpad(q), unpad(mxf), unpad(mnf),
                         degf, W, b.reshape(1, _C), relu)
    return h


# speedup vs baseline: 1.0272x; 1.0272x over previous
import functools
import numpy as np
import jax
import jax.numpy as jnp
from jax import lax
from jax.experimental import pallas as pl
from jax.experimental.pallas import tpu as pltpu
from jax.experimental.pallas import tpu_sc as plsc

_N = 10000
_E = 320000
_C = 128
_DEG = 32
_DELTA = float(np.log(_DEG + 1.0))

_NC = 2
_NS = 16
_L = 16
_NW = _NC * _NS

_NB = 64
_BRNG = 157
_BRP = 160
_GARB = _BRNG
_NPAD = _NB * _BRNG
_WRNG = 2 * _BRNG
_WRP = 320
_CH = 8000
_STG = _CH + 16
_K = 128
_NBUF = 2
_IB = 4096
_ECAP = _E + 16384
_BIG = 3.0e38

_ROWS = 400

_mesh = plsc.VectorSubcoreMesh(core_axis_name="c", subcore_axis_name="s")
_params = pltpu.CompilerParams(needs_layout_passes=False,
                               use_tc_tiling_on_sc=False)


def _bin_body(src_hbm, dst_hbm, lk_hbm, cnt_hbm, deg_hbm,
              srcvs, dstvs, st0, st1, degv, cntv, sems):
    cid = lax.axis_index("c")
    sid = lax.axis_index("s")
    w = cid * _NS + sid
    q0 = 2 * w
    q1 = 2 * w + 1

    zeros16f = jnp.zeros((_L,), jnp.float32)
    for j in range(_WRP // _L):
        degv[pl.ds(j * _L, _L)] = zeros16f

    ones16 = jnp.ones((_L,), jnp.float32)
    lanes = lax.iota(jnp.int32, _L)
    garb16 = jnp.full((_L,), _GARB, jnp.int32)

    def load_chunk(g, r):
        pltpu.async_copy(src_hbm.at[pl.ds(g * _CH, _CH)], srcvs[r], sems[2 * r])
        pltpu.async_copy(dst_hbm.at[pl.ds(g * _CH, _CH)], dstvs[r],
                         sems[2 * r + 1])

    def wait_chunk(g, r):
        pltpu.make_async_copy(src_hbm.at[pl.ds(g * _CH, _CH)], srcvs[r],
                              sems[2 * r]).wait()
        pltpu.make_async_copy(dst_hbm.at[pl.ds(g * _CH, _CH)], dstvs[r],
                              sems[2 * r + 1]).wait()

    def scan_chunk(r, tots):
        tot0, tot1 = tots
        srcv = srcvs[r]
        dstv = dstvs[r]

        def vreg_body(j, cnts):
            c0, c1 = cnts
            s = srcv[pl.ds(j * _L, _L)]
            d = dstv[pl.ds(j * _L, _L)]
            b = d // _BRNG
            dl = d - b * _BRNG
            key = s * 256 + dl
            m0 = b == q0
            m1 = b == q1
            plsc.store_compressed(st0.at[pl.ds(c0, _L)], key, mask=m0)
            plsc.store_compressed(st1.at[pl.ds(c1, _L)], key, mask=m1)
            didx = d - w * _WRNG
            plsc.addupdate_scatter(degv, [didx], ones16, mask=m0 | m1)
            return (c0 + plsc.all_reduce_population_count(m0)[0],
                    c1 + plsc.all_reduce_population_count(m1)[0])

        c0, c1 = plsc.parallel_loop(
            0, _CH // _L, carry=(jnp.int32(0), jnp.int32(0)))(vreg_body)
        p0 = (8 - (c0 % 8)) % 8
        plsc.store_compressed(st0.at[pl.ds(c0, _L)], garb16, mask=lanes < p0)
        c0 = c0 + p0
        p1 = (8 - (c1 % 8)) % 8
        plsc.store_compressed(st1.at[pl.ds(c1, _L)], garb16, mask=lanes < p1)
        c1 = c1 + p1
        f0 = pl.multiple_of(q0 * _ECAP + tot0, 8)
        pltpu.sync_copy(st0, lk_hbm.at[pl.ds(f0, _STG)])
        f1 = pl.multiple_of(q1 * _ECAP + tot1, 8)
        pltpu.sync_copy(st1, lk_hbm.at[pl.ds(f1, _STG)])
        return (tot0 + c0, tot1 + c1)

    def pair_body(t, tots):
        g = 2 * t
        load_chunk(g + 1, 1)
        wait_chunk(g, 0)
        tots = scan_chunk(0, tots)

        @pl.when(g + 2 < _E // _CH)
        def _next():
            load_chunk(g + 2, 0)

        wait_chunk(g + 1, 1)
        return scan_chunk(1, tots)

    load_chunk(0, 0)
    tot0, tot1 = lax.fori_loop(0, _E // (2 * _CH), pair_body,
                               (jnp.int32(0), jnp.int32(0)))

    for j in range(_NBUF * _K // _L):
        st0[pl.ds(j * _L, _L)] = garb16
    g0 = pl.multiple_of(q0 * _ECAP + tot0, 8)
    pltpu.sync_copy(st0.at[pl.ds(0, _NBUF * _K)],
                    lk_hbm.at[pl.ds(g0, _NBUF * _K)])
    g1 = pl.multiple_of(q1 * _ECAP + tot1, 8)
    pltpu.sync_copy(st0.at[pl.ds(0, _NBUF * _K)],
                    lk_hbm.at[pl.ds(g1, _NBUF * _K)])

    nb0 = (tot0 + _K - 1) // _K
    nb0 = ((nb0 + _NBUF - 1) // _NBUF) * _NBUF
    nb1 = (tot1 + _K - 1) // _K
    nb1 = ((nb1 + _NBUF - 1) // _NBUF) * _NBUF
    cntv[...] = jnp.where(lanes == 0, nb0, 0)
    pltpu.sync_copy(cntv, cnt_hbm.at[q0])
    cntv[...] = jnp.where(lanes == 0, nb1, 0)
    pltpu.sync_copy(cntv, cnt_hbm.at[q1])
    pltpu.sync_copy(degv, deg_hbm.at[w])


_bin_edges = functools.partial(
    pl.kernel,
    out_type=[
        jax.ShapeDtypeStruct((_NB * _ECAP,), jnp.int32),
        jax.ShapeDtypeStruct((_NB, _L), jnp.int32),
        jax.ShapeDtypeStruct((_NW, _WRP), jnp.float32),
    ],
    mesh=_mesh,
    compiler_params=_params,
    scratch_types=[
        [pltpu.VMEM((_CH,), jnp.int32) for _ in range(2)],
        [pltpu.VMEM((_CH,), jnp.int32) for _ in range(2)],
        pltpu.VMEM((_STG,), jnp.int32),
        pltpu.VMEM((_STG,), jnp.int32),
        pltpu.VMEM((_WRP,), jnp.float32),
        pltpu.VMEM((_L,), jnp.int32),
        [pltpu.SemaphoreType.DMA for _ in range(4)],
    ],
)(_bin_body)


def _agg_body(x_hbm, lk_hbm, cnt_hbm,
              sum_hbm, ssq_hbm, mx_hbm, mn_hbm,
              kblk, gbufs, sidxs, sacc, qacc, mxa, mna, cntv, sems):
    cid = lax.axis_index("c")
    sid = lax.axis_index("s")
    w = cid * _NS + sid

    posbig = jnp.full((_L,), _BIG, jnp.float32)
    negbig = jnp.full((_L,), -_BIG, jnp.float32)
    zeros16 = jnp.zeros((_L,), jnp.float32)
    m255 = jnp.full((_L,), 255, jnp.int32)

    def issue(kloc, r):
        for j in range(_K // _L):
            sidxs[r][pl.ds(j * _L, _L)] = lax.shift_right_logical(
                kblk[pl.ds(kloc * _K + j * _L, _L)], 8)
        pltpu.async_copy(x_hbm.at[sidxs[r]], gbufs[r], sems[r])

    def drain_rmw(kloc, r):
        pltpu.make_async_copy(x_hbm.at[sidxs[r]], gbufs[r], sems[r]).wait()
        gbuf = gbufs[r]

        def edge16(jj, _):
            dvec = kblk[pl.ds(kloc * _K + jj * _L, _L)] & m255
            roffs = [dvec[l] * _C for l in range(_L)]
            ibase = jj * _L

            def chunk_upd(c):
                co = c * _L
                for l in range(_L):
                    v = gbuf[ibase + l, pl.ds(co, _L)]
                    o = pl.ds(roffs[l] + co, _L)
                    sacc[o] = sacc[o] + v
                    qacc[o] = qacc[o] + v * v
                    mxa[o] = jnp.maximum(mxa[o], v)
                    mna[o] = jnp.minimum(mna[o], v)

            plsc.parallel_loop(0, _C // _L)(chunk_upd)
            return 0
        lax.fori_loop(0, _K // _L, edge16, 0)

    def bin_loop(sub, _):
        q = 2 * w + sub
        base = q * _ECAP

        def init_body(j, _2):
            o = pl.ds(j * _L, _L)
            sacc[o] = zeros16
            qacc[o] = zeros16
            mxa[o] = negbig
            mna[o] = posbig
            return 0
        lax.fori_loop(0, _BRP * _C // _L, init_body, 0)

        pltpu.sync_copy(cnt_hbm.at[q], cntv)
        nbp = cntv[...][0]

        def block_loop(tI, _2):
            boff = pl.multiple_of(base + tI * _IB, 8)
            pltpu.sync_copy(lk_hbm.at[pl.ds(boff, _IB)], kblk)
            nrem = jnp.minimum(nbp - tI * (_IB // _K), _IB // _K)

            @pl.when(nrem > 0)
            def _prologue():
                for r in range(_NBUF):
                    issue(r, r)

            def quad_loop(t, _3):
                k = _NBUF * t
                for r in range(_NBUF):
                    drain_rmw(k + r, r)

                    @pl.when(k + r + _NBUF < nrem)
                    def _next():
                        issue(k + r + _NBUF, r)
                return 0

            lax.fori_loop(0, nrem // _NBUF, quad_loop, 0)
            return 0

        lax.fori_loop(0, (nbp + _IB // _K - 1) // (_IB // _K), block_loop, 0)

        pltpu.sync_copy(sacc, sum_hbm.at[q])
        pltpu.sync_copy(qacc, ssq_hbm.at[q])
        pltpu.sync_copy(mxa, mx_hbm.at[q])
        pltpu.sync_copy(mna, mn_hbm.at[q])
        return 0

    lax.fori_loop(0, 2, bin_loop, 0)


_aggregate_sc = functools.partial(
    pl.kernel,
    out_type=[jax.ShapeDtypeStruct((_NB, _BRP * _C), jnp.float32)
              for _ in range(4)],
    mesh=_mesh,
    compiler_params=_params,
    scratch_types=[
        pltpu.VMEM((_IB,), jnp.int32),
        [pltpu.VMEM((_K, _C), jnp.float32) for _ in range(_NBUF)],
        [pltpu.VMEM((_K,), jnp.int32) for _ in range(_NBUF)],
        pltpu.VMEM((_BRP * _C,), jnp.float32),
        pltpu.VMEM((_BRP * _C,), jnp.float32),
        pltpu.VMEM((_BRP * _C,), jnp.float32),
        pltpu.VMEM((_BRP * _C,), jnp.float32),
        pltpu.VMEM((_L,), jnp.int32),
        [pltpu.SemaphoreType.DMA for _ in range(_NBUF)],
    ],
)(_agg_body)


def _dense_body(do_relu, x_ref, s_ref, q_ref, mx_ref, mn_ref, deg_ref,
                w_ref, b_ref, o_ref):
    deg = deg_ref[...]
    degc = jnp.maximum(deg, 1.0)
    inv = 1.0 / degc
    s = s_ref[...]
    mean = s * inv
    var = jnp.maximum(q_ref[...] * inv - mean * mean, 0.0)
    std = jnp.sqrt(var + 1e-5)
    has = deg > 0.0
    mx = jnp.where(has, mx_ref[...], 0.0)
    mn = jnp.where(has, mn_ref[...], 0.0)
    logd = jnp.log(deg + 1.0)
    amp = logd * (1.0 / _DELTA)
    att = _DELTA / jnp.clip(logd, 1e-5, None)

    agg = jnp.concatenate([mean, mn, mx, std], axis=1)
    w = w_ref[...]
    out = jnp.dot(x_ref[...], w[0:_C], preferred_element_type=jnp.float32)
    out += jnp.dot(agg, w[_C:5 * _C], preferred_element_type=jnp.float32)
    out += amp * jnp.dot(agg, w[5 * _C:9 * _C], preferred_element_type=jnp.float32)
    out += att * jnp.dot(agg, w[9 * _C:13 * _C], preferred_element_type=jnp.float32)
    out += b_ref[...]
    if do_relu:
        out = jnp.maximum(out, 0.0)
    o_ref[...] = out


def _dense_stage(x, s, q, mx, mn, degf, W, b, do_relu):
    grid = _N // _ROWS
    row_spec = pl.BlockSpec((_ROWS, _C), lambda i: (i, 0))
    out = pl.pallas_call(
        functools.partial(_dense_body, do_relu),
        grid=(grid,),
        in_specs=[
            row_spec, row_spec, row_spec, row_spec, row_spec,
            pl.BlockSpec((_ROWS, 1), lambda i: (i, 0)),
            pl.BlockSpec((13 * _C, _C), lambda i: (0, 0)),
            pl.BlockSpec((1, _C), lambda i: (0, 0)),
        ],
        out_specs=row_spec,
        out_shape=jax.ShapeDtypeStruct((_N, _C), jnp.float32),
    )(x, s, q, mx, mn, degf, W, b)
    return out


def kernel(x, edge_index, W0, b0, W1, b1, W2, b2):
    src = edge_index[0]
    dst = edge_index[1]

    lk, cnts, deg_rows = _bin_edges(src, dst)
    deg = deg_rows[:, :_WRNG].reshape(_NW * _WRNG)[:_N]
    degf = deg.reshape(_N, 1)

    def unpad(a):
        return a.reshape(_NB, _BRP, _C)[:, :_BRNG].reshape(_NPAD, _C)[:_N]

    h = x
    for W, b, relu in ((W0, b0, True), (W1, b1, True), (W2, b2, False)):
        s, q, mxf, mnf = _aggregate_sc(h, lk, cnts)
        h = _dense_stage(h, unpad(s), unpad(q), unpad(mxf), unpad(mnf),
                         degf, W, b.reshape(1, _C), relu)
    return h


# speedup vs baseline: 1.0330x; 1.0056x over previous
import functools
import numpy as np
import jax
import jax.numpy as jnp
from jax import lax
from jax.experimental import pallas as pl
from jax.experimental.pallas import tpu as pltpu
from jax.experimental.pallas import tpu_sc as plsc

_N = 10000
_E = 320000
_C = 128
_DEG = 32
_DELTA = float(np.log(_DEG + 1.0))

_NC = 2
_NS = 16
_L = 16
_NW = _NC * _NS

_NB = 64
_BRNG = 157
_BRP = 160
_GARB = _BRNG
_NPAD = _NB * _BRNG
_WRNG = 2 * _BRNG
_WRP = 320
_CH = 8000
_STG = _CH + 16
_K = 128
_NBUF = 2
_IB = 8192
_ECAP = _E + 16384
_BIG = 3.0e38

_ROWS = 400

_mesh = plsc.VectorSubcoreMesh(core_axis_name="c", subcore_axis_name="s")
_params = pltpu.CompilerParams(needs_layout_passes=False,
                               use_tc_tiling_on_sc=False)


def _bin_body(src_hbm, dst_hbm, lk_hbm, cnt_hbm, deg_hbm,
              srcvs, dstvs, st0, st1, degv, cntv, sems):
    cid = lax.axis_index("c")
    sid = lax.axis_index("s")
    w = cid * _NS + sid
    q0 = 2 * w
    q1 = 2 * w + 1

    zeros16f = jnp.zeros((_L,), jnp.float32)
    for j in range(_WRP // _L):
        degv[pl.ds(j * _L, _L)] = zeros16f

    ones16 = jnp.ones((_L,), jnp.float32)
    lanes = lax.iota(jnp.int32, _L)
    garb16 = jnp.full((_L,), _GARB, jnp.int32)

    def load_chunk(g, r):
        pltpu.async_copy(src_hbm.at[pl.ds(g * _CH, _CH)], srcvs[r], sems[2 * r])
        pltpu.async_copy(dst_hbm.at[pl.ds(g * _CH, _CH)], dstvs[r],
                         sems[2 * r + 1])

    def wait_chunk(g, r):
        pltpu.make_async_copy(src_hbm.at[pl.ds(g * _CH, _CH)], srcvs[r],
                              sems[2 * r]).wait()
        pltpu.make_async_copy(dst_hbm.at[pl.ds(g * _CH, _CH)], dstvs[r],
                              sems[2 * r + 1]).wait()

    def scan_chunk(r, tots):
        tot0, tot1 = tots
        srcv = srcvs[r]
        dstv = dstvs[r]

        def vreg_body(j, cnts):
            c0, c1 = cnts
            s = srcv[pl.ds(j * _L, _L)]
            d = dstv[pl.ds(j * _L, _L)]
            b = d // _BRNG
            dl = d - b * _BRNG
            key = s * 256 + dl
            m0 = b == q0
            m1 = b == q1
            plsc.store_compressed(st0.at[pl.ds(c0, _L)], key, mask=m0)
            plsc.store_compressed(st1.at[pl.ds(c1, _L)], key, mask=m1)
            didx = d - w * _WRNG
            plsc.addupdate_scatter(degv, [didx], ones16, mask=m0 | m1)
            return (c0 + plsc.all_reduce_population_count(m0)[0],
                    c1 + plsc.all_reduce_population_count(m1)[0])

        c0, c1 = plsc.parallel_loop(
            0, _CH // _L, carry=(jnp.int32(0), jnp.int32(0)))(vreg_body)
        p0 = (8 - (c0 % 8)) % 8
        plsc.store_compressed(st0.at[pl.ds(c0, _L)], garb16, mask=lanes < p0)
        c0 = c0 + p0
        p1 = (8 - (c1 % 8)) % 8
        plsc.store_compressed(st1.at[pl.ds(c1, _L)], garb16, mask=lanes < p1)
        c1 = c1 + p1
        f0 = pl.multiple_of(q0 * _ECAP + tot0, 8)
        pltpu.sync_copy(st0, lk_hbm.at[pl.ds(f0, _STG)])
        f1 = pl.multiple_of(q1 * _ECAP + tot1, 8)
        pltpu.sync_copy(st1, lk_hbm.at[pl.ds(f1, _STG)])
        return (tot0 + c0, tot1 + c1)

    def pair_body(t, tots):
        g = 2 * t
        load_chunk(g + 1, 1)
        wait_chunk(g, 0)
        tots = scan_chunk(0, tots)

        @pl.when(g + 2 < _E // _CH)
        def _next():
            load_chunk(g + 2, 0)

        wait_chunk(g + 1, 1)
        return scan_chunk(1, tots)

    load_chunk(0, 0)
    tot0, tot1 = lax.fori_loop(0, _E // (2 * _CH), pair_body,
                               (jnp.int32(0), jnp.int32(0)))

    for j in range(_NBUF * _K // _L):
        st0[pl.ds(j * _L, _L)] = garb16
    g0 = pl.multiple_of(q0 * _ECAP + tot0, 8)
    pltpu.sync_copy(st0.at[pl.ds(0, _NBUF * _K)],
                    lk_hbm.at[pl.ds(g0, _NBUF * _K)])
    g1 = pl.multiple_of(q1 * _ECAP + tot1, 8)
    pltpu.sync_copy(st0.at[pl.ds(0, _NBUF * _K)],
                    lk_hbm.at[pl.ds(g1, _NBUF * _K)])

    nb0 = (tot0 + _K - 1) // _K
    nb0 = ((nb0 + _NBUF - 1) // _NBUF) * _NBUF
    nb1 = (tot1 + _K - 1) // _K
    nb1 = ((nb1 + _NBUF - 1) // _NBUF) * _NBUF
    cntv[...] = jnp.where(lanes == 0, nb0, 0)
    pltpu.sync_copy(cntv, cnt_hbm.at[q0])
    cntv[...] = jnp.where(lanes == 0, nb1, 0)
    pltpu.sync_copy(cntv, cnt_hbm.at[q1])
    pltpu.sync_copy(degv, deg_hbm.at[w])


_bin_edges = functools.partial(
    pl.kernel,
    out_type=[
        jax.ShapeDtypeStruct((_NB * _ECAP,), jnp.int32),
        jax.ShapeDtypeStruct((_NB, _L), jnp.int32),
        jax.ShapeDtypeStruct((_NW, _WRP), jnp.float32),
    ],
    mesh=_mesh,
    compiler_params=_params,
    scratch_types=[
        [pltpu.VMEM((_CH,), jnp.int32) for _ in range(2)],
        [pltpu.VMEM((_CH,), jnp.int32) for _ in range(2)],
        pltpu.VMEM((_STG,), jnp.int32),
        pltpu.VMEM((_STG,), jnp.int32),
        pltpu.VMEM((_WRP,), jnp.float32),
        pltpu.VMEM((_L,), jnp.int32),
        [pltpu.SemaphoreType.DMA for _ in range(4)],
    ],
)(_bin_body)


def _agg_body(x_hbm, lk_hbm, cnt_hbm,
              sum_hbm, ssq_hbm, mx_hbm, mn_hbm,
              kblk, gbufs, sidxs, sacc, qacc, mxa, mna, cntv, sems):
    cid = lax.axis_index("c")
    sid = lax.axis_index("s")
    w = cid * _NS + sid

    posbig = jnp.full((_L,), _BIG, jnp.float32)
    negbig = jnp.full((_L,), -_BIG, jnp.float32)
    zeros16 = jnp.zeros((_L,), jnp.float32)
    m255 = jnp.full((_L,), 255, jnp.int32)

    def issue(kloc, r):
        for j in range(_K // _L):
            sidxs[r][pl.ds(j * _L, _L)] = lax.shift_right_logical(
                kblk[pl.ds(kloc * _K + j * _L, _L)], 8)
        pltpu.async_copy(x_hbm.at[sidxs[r]], gbufs[r], sems[r])

    def drain_rmw(kloc, r):
        pltpu.make_async_copy(x_hbm.at[sidxs[r]], gbufs[r], sems[r]).wait()
        gbuf = gbufs[r]

        def edge16(jj, _):
            dvec = kblk[pl.ds(kloc * _K + jj * _L, _L)] & m255
            roffs = [dvec[l] * _C for l in range(_L)]
            ibase = jj * _L

            def chunk_upd(c):
                co = c * _L
                for l in range(_L):
                    v = gbuf[ibase + l, pl.ds(co, _L)]
                    o = pl.ds(roffs[l] + co, _L)
                    sacc[o] = sacc[o] + v
                    qacc[o] = qacc[o] + v * v
                    mxa[o] = jnp.maximum(mxa[o], v)
                    mna[o] = jnp.minimum(mna[o], v)

            plsc.parallel_loop(0, _C // _L)(chunk_upd)
            return 0
        lax.fori_loop(0, _K // _L, edge16, 0)

    def bin_loop(sub, _):
        q = 2 * w + sub
        base = q * _ECAP

        def init_body(j, _2):
            o = pl.ds(j * _L, _L)
            sacc[o] = zeros16
            qacc[o] = zeros16
            mxa[o] = negbig
            mna[o] = posbig
            return 0
        lax.fori_loop(0, _BRP * _C // _L, init_body, 0)

        pltpu.sync_copy(cnt_hbm.at[q], cntv)
        nbp = cntv[...][0]

        def block_loop(tI, _2):
            boff = pl.multiple_of(base + tI * _IB, 8)
            pltpu.sync_copy(lk_hbm.at[pl.ds(boff, _IB)], kblk)
            nrem = jnp.minimum(nbp - tI * (_IB // _K), _IB // _K)

            @pl.when(nrem > 0)
            def _prologue():
                for r in range(_NBUF):
                    issue(r, r)

            def quad_loop(t, _3):
                k = _NBUF * t
                for r in range(_NBUF):
                    drain_rmw(k + r, r)

                    @pl.when(k + r + _NBUF < nrem)
                    def _next():
                        issue(k + r + _NBUF, r)
                return 0

            lax.fori_loop(0, nrem // _NBUF, quad_loop, 0)
            return 0

        lax.fori_loop(0, (nbp + _IB // _K - 1) // (_IB // _K), block_loop, 0)

        pltpu.sync_copy(sacc, sum_hbm.at[q])
        pltpu.sync_copy(qacc, ssq_hbm.at[q])
        pltpu.sync_copy(mxa, mx_hbm.at[q])
        pltpu.sync_copy(mna, mn_hbm.at[q])
        return 0

    lax.fori_loop(0, 2, bin_loop, 0)


_aggregate_sc = functools.partial(
    pl.kernel,
    out_type=[jax.ShapeDtypeStruct((_NB, _BRP * _C), jnp.float32)
              for _ in range(4)],
    mesh=_mesh,
    compiler_params=_params,
    scratch_types=[
        pltpu.VMEM((_IB,), jnp.int32),
        [pltpu.VMEM((_K, _C), jnp.float32) for _ in range(_NBUF)],
        [pltpu.VMEM((_K,), jnp.int32) for _ in range(_NBUF)],
        pltpu.VMEM((_BRP * _C,), jnp.float32),
        pltpu.VMEM((_BRP * _C,), jnp.float32),
        pltpu.VMEM((_BRP * _C,), jnp.float32),
        pltpu.VMEM((_BRP * _C,), jnp.float32),
        pltpu.VMEM((_L,), jnp.int32),
        [pltpu.SemaphoreType.DMA for _ in range(_NBUF)],
    ],
)(_agg_body)


def _dense_body(do_relu, x_ref, s_ref, q_ref, mx_ref, mn_ref, deg_ref,
                w_ref, b_ref, o_ref):
    deg = deg_ref[...]
    degc = jnp.maximum(deg, 1.0)
    inv = 1.0 / degc
    s = s_ref[...]
    mean = s * inv
    var = jnp.maximum(q_ref[...] * inv - mean * mean, 0.0)
    std = jnp.sqrt(var + 1e-5)
    has = deg > 0.0
    mx = jnp.where(has, mx_ref[...], 0.0)
    mn = jnp.where(has, mn_ref[...], 0.0)
    logd = jnp.log(deg + 1.0)
    amp = logd * (1.0 / _DELTA)
    att = _DELTA / jnp.clip(logd, 1e-5, None)

    agg = jnp.concatenate([mean, mn, mx, std], axis=1)
    w = w_ref[...]
    out = jnp.dot(x_ref[...], w[0:_C], preferred_element_type=jnp.float32)
    out += jnp.dot(agg, w[_C:5 * _C], preferred_element_type=jnp.float32)
    out += amp * jnp.dot(agg, w[5 * _C:9 * _C], preferred_element_type=jnp.float32)
    out += att * jnp.dot(agg, w[9 * _C:13 * _C], preferred_element_type=jnp.float32)
    out += b_ref[...]
    if do_relu:
        out = jnp.maximum(out, 0.0)
    o_ref[...] = out


def _dense_stage(x, s, q, mx, mn, degf, W, b, do_relu):
    grid = _N // _ROWS
    row_spec = pl.BlockSpec((_ROWS, _C), lambda i: (i, 0))
    out = pl.pallas_call(
        functools.partial(_dense_body, do_relu),
        grid=(grid,),
        in_specs=[
            row_spec, row_spec, row_spec, row_spec, row_spec,
            pl.BlockSpec((_ROWS, 1), lambda i: (i, 0)),
            pl.BlockSpec((13 * _C, _C), lambda i: (0, 0)),
            pl.BlockSpec((1, _C), lambda i: (0, 0)),
        ],
        out_specs=row_spec,
        out_shape=jax.ShapeDtypeStruct((_N, _C), jnp.float32),
    )(x, s, q, mx, mn, degf, W, b)
    return out


def kernel(x, edge_index, W0, b0, W1, b1, W2, b2):
    src = edge_index[0]
    dst = edge_index[1]

    lk, cnts, deg_rows = _bin_edges(src, dst)
    deg = deg_rows[:, :_WRNG].reshape(_NW * _WRNG)[:_N]
    degf = deg.reshape(_N, 1)

    def unpad(a):
        return a.reshape(_NB, _BRP, _C)[:, :_BRNG].reshape(_NPAD, _C)[:_N]

    h = x
    for W, b, relu in ((W0, b0, True), (W1, b1, True), (W2, b2, False)):
        s, q, mxf, mnf = _aggregate_sc(h, lk, cnts)
        h = _dense_stage(h, unpad(s), unpad(q), unpad(mxf), unpad(mnf),
                         degf, W, b.reshape(1, _C), relu)
    return h
